# baseline (speedup 1.0000x reference)
import functools

import jax
import jax.numpy as jnp
from jax import lax
from jax.experimental import pallas as pl
from jax.experimental.pallas import tpu as pltpu
from jax.experimental.pallas import tpu_sc as plsc

_N = 8388608
_BINS_MAX = 128
_NC = 2
_NS = 16
_NW = _NC * _NS
_L = 16
_PER_TILE = _N // _NW
_CHUNK = 8192
_NCHUNK = _PER_TILE // _CHUNK
_NB = _BINS_MAX * _L
_UNROLL = 8
_NSLOT = 4

_mesh = plsc.VectorSubcoreMesh(core_axis_name="c", subcore_axis_name="s")


@functools.partial(
    pl.kernel,
    out_type=jax.ShapeDtypeStruct((_NW, 2, _L), jnp.float32),
    mesh=_mesh,
    scratch_types=[
        pltpu.VMEM((2, _CHUNK), jnp.float32),
        pltpu.VMEM((2, _CHUNK), jnp.float32),
        pltpu.VMEM((2, _L), jnp.float32),
        pltpu.SemaphoreType.DMA,
        pltpu.SemaphoreType.DMA,
    ],
    compiler_params=pltpu.CompilerParams(needs_layout_passes=False),
)
def _minmax_kernel(sim_hbm, exp_hbm, out_hbm, sbuf, ebuf, obuf, sem0, sem1):
    wid = lax.axis_index("s") * _NC + lax.axis_index("c")
    base = wid * _PER_TILE
    sems = (sem0, sem1)

    def start(c, slot):
        off = pl.multiple_of(base + c * _CHUNK, _CHUNK)
        pltpu.async_copy(sim_hbm.at[pl.ds(off, _CHUNK)], sbuf.at[slot], sems[slot])
        pltpu.async_copy(exp_hbm.at[pl.ds(off, _CHUNK)], ebuf.at[slot], sems[slot])

    def wait(slot):
        pltpu.make_async_copy(sim_hbm.at[pl.ds(0, _CHUNK)], sbuf.at[slot], sems[slot]).wait()
        pltpu.make_async_copy(exp_hbm.at[pl.ds(0, _CHUNK)], ebuf.at[slot], sems[slot]).wait()

    start(0, 0)
    start(1, 1)

    def process(c, slot, carry):
        wait(slot)

        @plsc.parallel_loop(0, _CHUNK // _L, unroll=_UNROLL, carry=carry)
        def vec_body(i, carry2):
            mnv2, mxv2 = carry2
            sv = sbuf[slot, pl.ds(i * _L, _L)]
            ev = ebuf[slot, pl.ds(i * _L, _L)]
            return (jnp.minimum(mnv2, jnp.minimum(sv, ev)),
                    jnp.maximum(mxv2, jnp.maximum(sv, ev)))

        carry = vec_body

        @pl.when(c + 2 < _NCHUNK)
        def _():
            start(c + 2, slot)

        return carry

    def pair_body(p, carry):
        c0 = p * 2
        carry = process(c0, 0, carry)
        carry = process(c0 + 1, 1, carry)
        return carry

    init = (jnp.full((_L,), jnp.inf, jnp.float32),
            jnp.full((_L,), -jnp.inf, jnp.float32))
    mnv, mxv = lax.fori_loop(0, _NCHUNK // 2, pair_body, init)
    obuf[0, :] = mnv
    obuf[1, :] = mxv
    pltpu.sync_copy(obuf, out_hbm.at[wid])


@functools.partial(
    pl.kernel,
    out_type=tuple(jax.ShapeDtypeStruct((_NW, _NB), jnp.float32) for _ in range(4)),
    mesh=_mesh,
    scratch_types=[
        pltpu.VMEM((_NSLOT, _CHUNK), jnp.float32),
        pltpu.VMEM((_NSLOT, _CHUNK), jnp.float32),
        pltpu.VMEM((_NSLOT, _CHUNK), jnp.float32),
        pltpu.VMEM((48,), jnp.float32),
        pltpu.VMEM((_NB,), jnp.float32),
        pltpu.VMEM((_NB,), jnp.float32),
        pltpu.VMEM((_NB,), jnp.float32),
        pltpu.VMEM((_NB,), jnp.float32),
        pltpu.SemaphoreType.DMA,
        pltpu.SemaphoreType.DMA,
        pltpu.SemaphoreType.DMA,
        pltpu.SemaphoreType.DMA,
    ],
    compiler_params=pltpu.CompilerParams(needs_layout_passes=False),
)
def _hist_kernel(sim_hbm, exp_hbm, w_hbm, par_hbm,
                 outCs_hbm, outAs_hbm, outCe_hbm, outAe_hbm,
                 sbuf, ebuf, wbuf, pbuf, accCs, accAs, accCe, accAe,
                 sem0, sem1, sem2, sem3):
    wid = lax.axis_index("s") * _NC + lax.axis_index("c")
    base = wid * _PER_TILE
    sems = (sem0, sem1, sem2, sem3)

    pltpu.sync_copy(par_hbm, pbuf)
    mn16 = pbuf[pl.ds(0, _L)]
    sc16 = pbuf[pl.ds(16, _L)]
    st16 = pbuf[pl.ds(32, _L)]
    lane = lax.broadcasted_iota(jnp.int32, (_L,), 0)
    ones = jnp.full((_L,), 1.0, jnp.float32)
    zeros = jnp.zeros((_L,), jnp.float32)

    def zero_body(j, _):
        accCs[pl.ds(j * _L, _L)] = zeros
        accAs[pl.ds(j * _L, _L)] = zeros
        accCe[pl.ds(j * _L, _L)] = zeros
        accAe[pl.ds(j * _L, _L)] = zeros
        return 0

    lax.fori_loop(0, _BINS_MAX, zero_body, 0)

    def start(c, slot):
        off = pl.multiple_of(base + c * _CHUNK, _CHUNK)
        pltpu.async_copy(sim_hbm.at[pl.ds(off, _CHUNK)], sbuf.at[slot], sems[slot])
        pltpu.async_copy(exp_hbm.at[pl.ds(off, _CHUNK)], ebuf.at[slot], sems[slot])
        pltpu.async_copy(w_hbm.at[pl.ds(off, _CHUNK)], wbuf.at[slot], sems[slot])

    def wait(slot):
        pltpu.make_async_copy(sim_hbm.at[pl.ds(0, _CHUNK)], sbuf.at[slot], sems[slot]).wait()
        pltpu.make_async_copy(exp_hbm.at[pl.ds(0, _CHUNK)], ebuf.at[slot], sems[slot]).wait()
        pltpu.make_async_copy(w_hbm.at[pl.ds(0, _CHUNK)], wbuf.at[slot], sems[slot]).wait()

    for s in range(_NSLOT):
        start(s, s)

    def process(c, slot):
        wait(slot)

        @plsc.parallel_loop(0, _CHUNK // _L, unroll=_UNROLL)
        def vec_body(i):
            off = i * _L
            o_s = sbuf[slot, pl.ds(off, _L)]
            o_e = ebuf[slot, pl.ds(off, _L)]
            wv = wbuf[slot, pl.ds(off, _L)]
            ks = jnp.clip(((o_s - mn16) * sc16).astype(jnp.int32), 0, _BINS_MAX - 1)
            ke = jnp.clip(((o_e - mn16) * sc16).astype(jnp.int32), 0, _BINS_MAX - 1)
            gs = mn16 + ks.astype(jnp.float32) * st16
            ge = mn16 + ke.astype(jnp.float32) * st16
            addr_s = ks * _L + lane
            addr_e = ke * _L + lane
            plsc.addupdate_scatter(accCs, [addr_s], wv)
            plsc.addupdate_scatter(accAs, [addr_s], wv * (o_s - gs))
            plsc.addupdate_scatter(accCe, [addr_e], ones)
            plsc.addupdate_scatter(accAe, [addr_e], o_e - ge)

        @pl.when(c + _NSLOT < _NCHUNK)
        def _():
            start(c + _NSLOT, slot)

    def ring_body(p, _):
        for s in range(_NSLOT):
            process(p * _NSLOT + s, s)
        return 0

    lax.fori_loop(0, _NCHUNK // _NSLOT, ring_body, 0)

    pltpu.sync_copy(accCs, outCs_hbm.at[wid])
    pltpu.sync_copy(accAs, outAs_hbm.at[wid])
    pltpu.sync_copy(accCe, outCe_hbm.at[wid])
    pltpu.sync_copy(accAe, outAe_hbm.at[wid])


def kernel(sim_observable, exp_observable, weights):
    mm = _minmax_kernel(sim_observable, exp_observable)
    mn = jnp.min(mm[:, 0, :])
    mx = jnp.max(mm[:, 1, :])
    bins = (mx - mn).astype(jnp.int32)
    bins_f = bins.astype(jnp.float32)
    span = mx - mn
    denom = bins_f - jnp.float32(1)
    delta = span / bins_f
    scale = denom / span
    step = span / denom

    par = jnp.concatenate([
        jnp.full((_L,), mn, jnp.float32),
        jnp.full((_L,), scale, jnp.float32),
        jnp.full((_L,), step, jnp.float32),
    ])
    pCs, pAs, pCe, pAe = _hist_kernel(sim_observable, exp_observable, weights, par)

    def tot(p):
        return jnp.sum(p.reshape(_NW, _BINS_MAX, _L), axis=(0, 2))

    Cs, As, Ce, Ae = tot(pCs), tot(pAs), tot(pCe), tot(pAe)

    d = jnp.arange(_BINS_MAX)
    df = d.astype(jnp.float32)
    h = mn + span * df / denom
    g = mn + df * step

    def hist(C, A):
        z1 = jnp.zeros((1,), jnp.float32)
        Am = jnp.concatenate([z1, A[:-1]])
        Cm = jnp.concatenate([z1, C[:-1]])
        gm = jnp.concatenate([z1, g[:-1]])
        hm = jnp.concatenate([z1, h[:-1]])
        hp = jnp.concatenate([h[1:], z1])
        v = (Am + Cm * (gm - hm)) + (C * (hp - g) - A)
        valid = (d >= 1) & (d <= _BINS_MAX - 2) & (d < bins - 1)
        hh = jnp.where(valid, v, jnp.float32(0))
        hh = hh / jnp.sum(hh)
        return hh / delta

    hs = hist(Cs, As)
    he = hist(Ce, Ae)
    return jnp.sum((hs - he) ** 2)


# speedup vs baseline: 1.0392x; 1.0392x over previous
import functools

import jax
import jax.numpy as jnp
from jax import lax
from jax.experimental import pallas as pl
from jax.experimental.pallas import tpu as pltpu
from jax.experimental.pallas import tpu_sc as plsc

_N = 8388608
_BINS_MAX = 128
_NC = 2
_NS = 16
_NW = _NC * _NS
_L = 16
_PER_TILE = _N // _NW
_CHUNK = 8192
_NCHUNK = _PER_TILE // _CHUNK
_NB = _BINS_MAX * _L
_UNROLL = 4
_NSLOT = 4

_mesh = plsc.VectorSubcoreMesh(core_axis_name="c", subcore_axis_name="s")


@functools.partial(
    pl.kernel,
    out_type=jax.ShapeDtypeStruct((_NW, 2, _L), jnp.float32),
    mesh=_mesh,
    scratch_types=[
        pltpu.VMEM((2, _CHUNK), jnp.float32),
        pltpu.VMEM((2, _CHUNK), jnp.float32),
        pltpu.VMEM((2, _L), jnp.float32),
        pltpu.SemaphoreType.DMA,
        pltpu.SemaphoreType.DMA,
    ],
    compiler_params=pltpu.CompilerParams(needs_layout_passes=False),
)
def _minmax_kernel(sim_hbm, exp_hbm, out_hbm, sbuf, ebuf, obuf, sem0, sem1):
    wid = lax.axis_index("s") * _NC + lax.axis_index("c")
    base = wid * _PER_TILE
    sems = (sem0, sem1)

    def start(c, slot):
        off = pl.multiple_of(base + c * _CHUNK, _CHUNK)
        pltpu.async_copy(sim_hbm.at[pl.ds(off, _CHUNK)], sbuf.at[slot], sems[slot])
        pltpu.async_copy(exp_hbm.at[pl.ds(off, _CHUNK)], ebuf.at[slot], sems[slot])

    def wait(slot):
        pltpu.make_async_copy(sim_hbm.at[pl.ds(0, _CHUNK)], sbuf.at[slot], sems[slot]).wait()
        pltpu.make_async_copy(exp_hbm.at[pl.ds(0, _CHUNK)], ebuf.at[slot], sems[slot]).wait()

    start(0, 0)
    start(1, 1)

    def process(c, slot, carry):
        wait(slot)

        @plsc.parallel_loop(0, _CHUNK // _L, unroll=_UNROLL, carry=carry)
        def vec_body(i, carry2):
            mnv2, mxv2 = carry2
            sv = sbuf[slot, pl.ds(i * _L, _L)]
            ev = ebuf[slot, pl.ds(i * _L, _L)]
            return (jnp.minimum(mnv2, jnp.minimum(sv, ev)),
                    jnp.maximum(mxv2, jnp.maximum(sv, ev)))

        carry = vec_body

        @pl.when(c + 2 < _NCHUNK)
        def _():
            start(c + 2, slot)

        return carry

    def pair_body(p, carry):
        c0 = p * 2
        carry = process(c0, 0, carry)
        carry = process(c0 + 1, 1, carry)
        return carry

    init = (jnp.full((_L,), jnp.inf, jnp.float32),
            jnp.full((_L,), -jnp.inf, jnp.float32))
    mnv, mxv = lax.fori_loop(0, _NCHUNK // 2, pair_body, init)
    obuf[0, :] = mnv
    obuf[1, :] = mxv
    pltpu.sync_copy(obuf, out_hbm.at[wid])


@functools.partial(
    pl.kernel,
    out_type=tuple(jax.ShapeDtypeStruct((_NW, _NB), jnp.float32) for _ in range(4)),
    mesh=_mesh,
    scratch_types=[
        pltpu.VMEM((_NSLOT, _CHUNK), jnp.float32),
        pltpu.VMEM((_NSLOT, _CHUNK), jnp.float32),
        pltpu.VMEM((_NSLOT, _CHUNK), jnp.float32),
        pltpu.VMEM((_NW, 2, _L), jnp.float32),
        pltpu.VMEM((_NB,), jnp.float32),
        pltpu.VMEM((_NB,), jnp.float32),
        pltpu.VMEM((_NB,), jnp.float32),
        pltpu.VMEM((_NB,), jnp.float32),
        pltpu.SemaphoreType.DMA,
        pltpu.SemaphoreType.DMA,
        pltpu.SemaphoreType.DMA,
        pltpu.SemaphoreType.DMA,
    ],
    compiler_params=pltpu.CompilerParams(needs_layout_passes=False),
)
def _hist_kernel(sim_hbm, exp_hbm, w_hbm, mm_hbm,
                 outCs_hbm, outAs_hbm, outCe_hbm, outAe_hbm,
                 sbuf, ebuf, wbuf, mmbuf, accCs, accAs, accCe, accAe,
                 sem0, sem1, sem2, sem3):
    wid = lax.axis_index("s") * _NC + lax.axis_index("c")
    base = wid * _PER_TILE
    sems = (sem0, sem1, sem2, sem3)

    pltpu.sync_copy(mm_hbm, mmbuf)
    mnv = mmbuf[0, 0, :]
    mxv = mmbuf[0, 1, :]
    for t in range(1, _NW):
        mnv = jnp.minimum(mnv, mmbuf[t, 0, :])
        mxv = jnp.maximum(mxv, mmbuf[t, 1, :])
    mn = lax.reduce_min(mnv, (0,))
    mx = lax.reduce_max(mxv, (0,))
    mn16 = jnp.full((_L,), mn, jnp.float32)
    span16 = jnp.full((_L,), mx - mn, jnp.float32)
    bins_f16 = (span16.astype(jnp.int32)).astype(jnp.float32)
    denom16 = bins_f16 - jnp.float32(1)
    sc16 = denom16 / span16
    st16 = span16 / denom16
    lane = lax.broadcasted_iota(jnp.int32, (_L,), 0)
    ones = jnp.full((_L,), 1.0, jnp.float32)
    zeros = jnp.zeros((_L,), jnp.float32)

    def zero_body(j, _):
        accCs[pl.ds(j * _L, _L)] = zeros
        accAs[pl.ds(j * _L, _L)] = zeros
        accCe[pl.ds(j * _L, _L)] = zeros
        accAe[pl.ds(j * _L, _L)] = zeros
        return 0

    lax.fori_loop(0, _BINS_MAX, zero_body, 0)

    def start(c, slot):
        off = pl.multiple_of(base + c * _CHUNK, _CHUNK)
        pltpu.async_copy(sim_hbm.at[pl.ds(off, _CHUNK)], sbuf.at[slot], sems[slot])
        pltpu.async_copy(exp_hbm.at[pl.ds(off, _CHUNK)], ebuf.at[slot], sems[slot])
        pltpu.async_copy(w_hbm.at[pl.ds(off, _CHUNK)], wbuf.at[slot], sems[slot])

    def wait(slot):
        pltpu.make_async_copy(sim_hbm.at[pl.ds(0, _CHUNK)], sbuf.at[slot], sems[slot]).wait()
        pltpu.make_async_copy(exp_hbm.at[pl.ds(0, _CHUNK)], ebuf.at[slot], sems[slot]).wait()
        pltpu.make_async_copy(w_hbm.at[pl.ds(0, _CHUNK)], wbuf.at[slot], sems[slot]).wait()

    for s in range(_NSLOT):
        start(s, s)

    def process(c, slot):
        wait(slot)

        @plsc.parallel_loop(0, _CHUNK // _L, unroll=_UNROLL)
        def vec_body(i):
            off = i * _L
            o_s = sbuf[slot, pl.ds(off, _L)]
            o_e = ebuf[slot, pl.ds(off, _L)]
            wv = wbuf[slot, pl.ds(off, _L)]
            ks = jnp.clip(((o_s - mn16) * sc16).astype(jnp.int32), 0, _BINS_MAX - 1)
            ke = jnp.clip(((o_e - mn16) * sc16).astype(jnp.int32), 0, _BINS_MAX - 1)
            gs = mn16 + ks.astype(jnp.float32) * st16
            ge = mn16 + ke.astype(jnp.float32) * st16
            addr_s = ks * _L + lane
            addr_e = ke * _L + lane
            plsc.addupdate_scatter(accCs, [addr_s], wv)
            plsc.addupdate_scatter(accAs, [addr_s], wv * (o_s - gs))
            plsc.addupdate_scatter(accCe, [addr_e], ones)
            plsc.addupdate_scatter(accAe, [addr_e], o_e - ge)

        @pl.when(c + _NSLOT < _NCHUNK)
        def _():
            start(c + _NSLOT, slot)

    def ring_body(p, _):
        for s in range(_NSLOT):
            process(p * _NSLOT + s, s)
        return 0

    lax.fori_loop(0, _NCHUNK // _NSLOT, ring_body, 0)

    pltpu.sync_copy(accCs, outCs_hbm.at[wid])
    pltpu.sync_copy(accAs, outAs_hbm.at[wid])
    pltpu.sync_copy(accCe, outCe_hbm.at[wid])
    pltpu.sync_copy(accAe, outAe_hbm.at[wid])


def kernel(sim_observable, exp_observable, weights):
    mm = _minmax_kernel(sim_observable, exp_observable)
    mn = jnp.min(mm[:, 0, :])
    mx = jnp.max(mm[:, 1, :])
    bins = (mx - mn).astype(jnp.int32)
    bins_f = bins.astype(jnp.float32)
    span = mx - mn
    denom = bins_f - jnp.float32(1)
    delta = span / bins_f
    step = span / denom

    pCs, pAs, pCe, pAe = _hist_kernel(sim_observable, exp_observable, weights, mm)

    def tot(p):
        return jnp.sum(p.reshape(_NW, _BINS_MAX, _L), axis=(0, 2))

    Cs, As, Ce, Ae = tot(pCs), tot(pAs), tot(pCe), tot(pAe)

    d = jnp.arange(_BINS_MAX)
    df = d.astype(jnp.float32)
    h = mn + span * df / denom
    g = mn + df * step

    def hist(C, A):
        z1 = jnp.zeros((1,), jnp.float32)
        Am = jnp.concatenate([z1, A[:-1]])
        Cm = jnp.concatenate([z1, C[:-1]])
        gm = jnp.concatenate([z1, g[:-1]])
        hm = jnp.concatenate([z1, h[:-1]])
        hp = jnp.concatenate([h[1:], z1])
        v = (Am + Cm * (gm - hm)) + (C * (hp - g) - A)
        valid = (d >= 1) & (d <= _BINS_MAX - 2) & (d < bins - 1)
        hh = jnp.where(valid, v, jnp.float32(0))
        hh = hh / jnp.sum(hh)
        return hh / delta

    hs = hist(Cs, As)
    he = hist(Ce, Ae)
    return jnp.sum((hs - he) ** 2)
